# Initial kernel scaffold; baseline (speedup 1.0000x reference)
import jax
import jax.numpy as jnp
from jax.experimental import pallas as pl


def kernel(x, edge_index, batch, emb, conv1_Wl, conv1_bl, conv1_Wr, pool1_w, conv2_Wl, conv2_bl, conv2_Wr, pool2_w, conv3_Wl, conv3_bl, conv3_Wr, pool3_w, lin1_W, lin1_b, lin2_W, lin2_b, lin3_W, lin3_b):
    raise NotImplementedError("write your pallas kernel here")


# speedup vs baseline: 14.1517x; 14.1517x over previous
import functools

import jax
import jax.numpy as jnp
from jax import lax
from jax.experimental import pallas as pl
from jax.experimental.pallas import tpu as pltpu
from jax.experimental.pallas import tpu_sc as plsc

N = 10000
E = 320000
H = 128
NW = 32
EPW = E // NW
CH = 128
NFULL = EPW // CH
TAIL = EPW - NFULL * CH
RPT = N // 16
RB = 1000
GRID = N // RB
NPAD = 10240
NEGINF = float("-inf")


def _sc_mesh():
    return plsc.VectorSubcoreMesh(core_axis_name="c", subcore_axis_name="s")


def _he_gather(emb_pad, xi):
    RW = 312
    GC = 104

    @functools.partial(
        pl.kernel,
        out_type=jax.ShapeDtypeStruct((N, 16), jnp.float32),
        mesh=_sc_mesh(),
        compiler_params=pltpu.CompilerParams(use_tc_tiling_on_sc=False),
        scratch_types=[
            pltpu.VMEM((GC,), jnp.int32),
            pltpu.VMEM((GC, 16), jnp.float32),
            pltpu.VMEM((16,), jnp.int32),
            pltpu.VMEM((16, 16), jnp.float32),
            pltpu.SemaphoreType.DMA,
        ],
    )
    def hek(emb_ref, xi_ref, out_ref, idx, rows, idxt, rowst, sem):
        c = lax.axis_index("c")
        s = lax.axis_index("s")
        wid = c * 16 + s
        base = wid * RW
        for i in range(RW // GC):
            off = base + i * GC
            pltpu.sync_copy(xi_ref.at[pl.ds(off, GC)], idx)
            pltpu.async_copy(emb_ref.at[idx], rows, sem).wait()
            pltpu.sync_copy(rows, out_ref.at[pl.ds(off, GC)])

        @pl.when(wid == 0)
        def _():
            pltpu.sync_copy(xi_ref.at[pl.ds(NW * RW, 16)], idxt)
            pltpu.async_copy(emb_ref.at[idxt], rowst, sem).wait()
            pltpu.sync_copy(rowst, out_ref.at[pl.ds(NW * RW, 16)])

    return hek(emb_pad, xi)


def _make_segsum(Dp):
    @functools.partial(
        pl.kernel,
        out_type=jax.ShapeDtypeStruct((2, N, Dp), jnp.float32),
        mesh=_sc_mesh(),
        compiler_params=pltpu.CompilerParams(use_tc_tiling_on_sc=False),
        scratch_types=[
            pltpu.VMEM((CH,), jnp.int32),
            pltpu.VMEM((CH,), jnp.int32),
            pltpu.VMEM((CH, Dp), jnp.float32),
            pltpu.VMEM((TAIL,), jnp.int32),
            pltpu.VMEM((TAIL,), jnp.int32),
            pltpu.VMEM((TAIL, Dp), jnp.float32),
            pltpu.VMEM_SHARED((N, Dp), jnp.float32),
            pltpu.SemaphoreType.DMA,
        ],
    )
    def seg(tab_ref, src_ref, dst_ref, out_ref,
            sbuf, dbuf, rows, sbuft, dbuft, rowst, acc, sem):
        c = lax.axis_index("c")
        s = lax.axis_index("s")
        wid = c * 16 + s

        def zrow(i, carry):
            def zcol(j, carry2):
                rows[i, pl.ds(j * 16, 16)] = jnp.zeros((16,), jnp.float32)
                return carry2
            return lax.fori_loop(0, Dp // 16, zcol, carry)
        lax.fori_loop(0, CH, zrow, 0)

        zbase = s * RPT
        for i in range(RPT // CH):
            pltpu.sync_copy(rows, acc.at[pl.ds(zbase + i * CH, CH)])
        rem = RPT - (RPT // CH) * CH
        pltpu.sync_copy(rows.at[pl.ds(0, rem)],
                        acc.at[pl.ds(zbase + (RPT // CH) * CH, rem)])
        plsc.subcore_barrier()

        ebase = wid * EPW

        def chunk(j, carry):
            off = ebase + j * CH
            pltpu.sync_copy(src_ref.at[pl.ds(off, CH)], sbuf)
            pltpu.sync_copy(dst_ref.at[pl.ds(off, CH)], dbuf)
            pltpu.async_copy(tab_ref.at[sbuf], rows, sem).wait()
            pltpu.sync_copy(rows, acc.at[dbuf], add=True)
            return carry
        lax.fori_loop(0, NFULL, chunk, 0)

        offt = ebase + NFULL * CH
        pltpu.sync_copy(src_ref.at[pl.ds(offt, TAIL)], sbuft)
        pltpu.sync_copy(dst_ref.at[pl.ds(offt, TAIL)], dbuft)
        pltpu.async_copy(tab_ref.at[sbuft], rowst, sem).wait()
        pltpu.sync_copy(rowst, acc.at[dbuft], add=True)

        plsc.subcore_barrier()
        pltpu.sync_copy(acc.at[pl.ds(s * RPT, RPT)],
                        out_ref.at[c, pl.ds(s * RPT, RPT)])

    return seg


_segsum16 = _make_segsum(16)
_segsum144 = _make_segsum(144)


def _dense_body(Din, parts_ref, f_ref, wl_ref, bl_ref, wr_ref, w_ref,
                h_ref, u_ref):
    p = parts_ref[0] + parts_ref[1]
    deg = jnp.maximum(p[:, Din:Din + 1], 1.0)
    mean = p[:, :Din] / deg
    xin = f_ref[:, :Din]
    h = mean @ wl_ref[...] + bl_ref[...] + xin @ wr_ref[...]
    h = jnp.maximum(h, 0.0)
    w = w_ref[...]
    u = (h @ w) * lax.rsqrt(jnp.sum(w * w))
    keep = f_ref[:, Din:Din + 1]
    h_ref[...] = h
    u_ref[...] = jnp.where(keep > 0, u, NEGINF)


def _dense(parts, F, Wl, bl, Wr, w, Din):
    Dp = F.shape[1]
    return pl.pallas_call(
        functools.partial(_dense_body, Din),
        grid=(GRID,),
        in_specs=[
            pl.BlockSpec((2, RB, Dp), lambda i: (0, i, 0)),
            pl.BlockSpec((RB, Dp), lambda i: (i, 0)),
            pl.BlockSpec(Wl.shape, lambda i: (0, 0)),
            pl.BlockSpec(bl.shape, lambda i: (0, 0)),
            pl.BlockSpec(Wr.shape, lambda i: (0, 0)),
            pl.BlockSpec(w.shape, lambda i: (0, 0)),
        ],
        out_specs=[
            pl.BlockSpec((RB, H), lambda i: (i, 0)),
            pl.BlockSpec((RB, 1), lambda i: (i, 0)),
        ],
        out_shape=[
            jax.ShapeDtypeStruct((N, H), jnp.float32),
            jax.ShapeDtypeStruct((N, 1), jnp.float32),
        ],
    )(parts, F, Wl, bl, Wr, w)


def _select_body(k, u_ref, g_ref, keep_ref):
    u = u_ref[...]
    b = lax.bitcast_convert_type(u, jnp.uint32)
    keys = jnp.where(u < 0, ~b, b | jnp.uint32(0x80000000))

    def it(i, prefix):
        shift = (jnp.uint32(31) - i.astype(jnp.uint32))
        cand = prefix | jnp.left_shift(jnp.uint32(1), shift)
        cnt = jnp.sum((keys >= cand).astype(jnp.int32))
        return jnp.where(cnt >= k, cand, prefix)

    thr = lax.fori_loop(0, 32, it, jnp.uint32(0))
    keep = (keys >= thr).astype(jnp.float32)
    g_ref[...] = jnp.tanh(u) * keep
    keep_ref[...] = keep


def _select(upad, k):
    return pl.pallas_call(
        functools.partial(_select_body, k),
        out_shape=[
            jax.ShapeDtypeStruct((NPAD // 128, 128), jnp.float32),
            jax.ShapeDtypeStruct((NPAD // 128, 128), jnp.float32),
        ],
    )(upad)


def _asm_body(h_ref, g_ref, keep_ref, f_ref, ro_ref):
    i = pl.program_id(0)
    g = g_ref[...]
    kp = keep_ref[...]
    hm = h_ref[...] * g
    f_ref[...] = jnp.concatenate(
        [hm, kp, jnp.zeros((RB, 15), jnp.float32)], axis=1)
    mx = jnp.max(jnp.where(kp > 0, hm, NEGINF), axis=0, keepdims=True)
    sm = jnp.sum(hm, axis=0, keepdims=True)

    @pl.when(i == 0)
    def _():
        ro_ref[...] = jnp.concatenate([mx, sm], axis=0)

    @pl.when(i > 0)
    def _():
        ro_ref[0:1, :] = jnp.maximum(ro_ref[0:1, :], mx)
        ro_ref[1:2, :] = ro_ref[1:2, :] + sm


def _assemble(h, g, kp):
    return pl.pallas_call(
        _asm_body,
        grid=(GRID,),
        in_specs=[
            pl.BlockSpec((RB, H), lambda i: (i, 0)),
            pl.BlockSpec((RB, 1), lambda i: (i, 0)),
            pl.BlockSpec((RB, 1), lambda i: (i, 0)),
        ],
        out_specs=[
            pl.BlockSpec((RB, 144), lambda i: (i, 0)),
            pl.BlockSpec((2, H), lambda i: (0, 0)),
        ],
        out_shape=[
            jax.ShapeDtypeStruct((N, 144), jnp.float32),
            jax.ShapeDtypeStruct((2, H), jnp.float32),
        ],
    )(h, g, kp)


def _mlp_body(z_ref, w1, b1, w2, b2, w3, b3, out_ref):
    z = z_ref[...]
    z = jnp.maximum(z @ w1[...] + b1[...], 0.0)
    z = jnp.maximum(z @ w2[...] + b2[...], 0.0)
    z = z @ w3[...] + b3[...]
    out_ref[...] = 1.0 / (1.0 + jnp.exp(-z))


def _mlp(z, W1, b1, W2, b2, W3, b3):
    return pl.pallas_call(
        _mlp_body,
        out_shape=jax.ShapeDtypeStruct((1, 1), jnp.float32),
    )(z, W1, b1, W2, b2, W3, b3)


def kernel(x, edge_index, batch, emb,
           conv1_Wl, conv1_bl, conv1_Wr, pool1_w,
           conv2_Wl, conv2_bl, conv2_Wr, pool2_w,
           conv3_Wl, conv3_bl, conv3_Wr, pool3_w,
           lin1_W, lin1_b, lin2_W, lin2_b, lin3_W, lin3_b):
    V = emb.shape[0]
    xi = x[:, 0]
    src = jnp.asarray(edge_index[0])
    dst = jnp.asarray(edge_index[1])

    emb_pad = jnp.concatenate(
        [emb, jnp.ones((V, 1), jnp.float32), jnp.zeros((V, 6), jnp.float32)],
        axis=1)

    F = _he_gather(emb_pad, xi)
    Din = 9
    seg = _segsum16
    layers = [
        (conv1_Wl, conv1_bl, conv1_Wr, pool1_w, 8000),
        (conv2_Wl, conv2_bl, conv2_Wr, pool2_w, 6400),
        (conv3_Wl, conv3_bl, conv3_Wr, pool3_w, 5120),
    ]
    zparts = []
    for Wl, bl, Wr, pw, k in layers:
        parts = seg(F, src, dst)
        h, u = _dense(parts, F, Wl, bl[None, :], Wr, pw[:, None], Din)
        upad = jnp.concatenate(
            [u[:, 0], jnp.full((NPAD - N,), NEGINF, jnp.float32)]
        ).reshape(NPAD // 128, 128)
        g80, k80 = _select(upad, k)
        g = g80.reshape(NPAD)[:N][:, None]
        kp = k80.reshape(NPAD)[:N][:, None]
        F, ro = _assemble(h, g, kp)
        zparts.append(jnp.concatenate([ro[0], ro[1] / k]))
        Din = 128
        seg = _segsum144

    z = (zparts[0] + zparts[1] + zparts[2])[None, :]
    out = _mlp(z, lin1_W, lin1_b[None, :], lin2_W, lin2_b[None, :],
               lin3_W, lin3_b[None, :])
    return out[:, 0]


# speedup vs baseline: 20.0902x; 1.4196x over previous
import functools

import jax
import jax.numpy as jnp
from jax import lax
from jax.experimental import pallas as pl
from jax.experimental.pallas import tpu as pltpu
from jax.experimental.pallas import tpu_sc as plsc

N = 10000
E = 320000
H = 128
NW = 32
EPW = E // NW
CH = 128
NFULL = EPW // CH
TAIL = EPW - NFULL * CH
RPT = N // 16
RB = 1000
GRID = N // RB
NPAD = 10240
NEGINF = float("-inf")


def _sc_mesh():
    return plsc.VectorSubcoreMesh(core_axis_name="c", subcore_axis_name="s")


def _he_gather(emb_pad, xi):
    RW = 312
    GC = 104

    @functools.partial(
        pl.kernel,
        out_type=jax.ShapeDtypeStruct((N, 16), jnp.float32),
        mesh=_sc_mesh(),
        compiler_params=pltpu.CompilerParams(use_tc_tiling_on_sc=False),
        scratch_types=[
            pltpu.VMEM((GC,), jnp.int32),
            pltpu.VMEM((GC, 16), jnp.float32),
            pltpu.VMEM((16,), jnp.int32),
            pltpu.VMEM((16, 16), jnp.float32),
            pltpu.SemaphoreType.DMA,
        ],
    )
    def hek(emb_ref, xi_ref, out_ref, idx, rows, idxt, rowst, sem):
        c = lax.axis_index("c")
        s = lax.axis_index("s")
        wid = c * 16 + s
        base = wid * RW
        for i in range(RW // GC):
            off = base + i * GC
            pltpu.sync_copy(xi_ref.at[pl.ds(off, GC)], idx)
            pltpu.async_copy(emb_ref.at[idx], rows, sem).wait()
            pltpu.sync_copy(rows, out_ref.at[pl.ds(off, GC)])

        @pl.when(wid == 0)
        def _():
            pltpu.sync_copy(xi_ref.at[pl.ds(NW * RW, 16)], idxt)
            pltpu.async_copy(emb_ref.at[idxt], rowst, sem).wait()
            pltpu.sync_copy(rowst, out_ref.at[pl.ds(NW * RW, 16)])

    return hek(emb_pad, xi)


def _make_segsum(Dp):
    UC = 4 if Dp == 16 else 1
    NUNITS = E // (UC * CH)
    NBASE = NUNITS // NW
    NEXTRA = NUNITS - NBASE * NW
    NMAX = NBASE + (1 if NEXTRA else 0)

    @functools.partial(
        pl.kernel,
        out_type=jax.ShapeDtypeStruct((2, N, Dp), jnp.float32),
        mesh=_sc_mesh(),
        compiler_params=pltpu.CompilerParams(use_tc_tiling_on_sc=False),
        scratch_types=[
            pltpu.VMEM((UC, CH), jnp.int32),
            [pltpu.VMEM((CH,), jnp.int32) for _ in range(UC)],
            pltpu.VMEM((UC, CH, Dp), jnp.float32),
            pltpu.VMEM((UC, CH), jnp.int32),
            [pltpu.VMEM((CH,), jnp.int32) for _ in range(UC)],
            pltpu.VMEM((UC, CH, Dp), jnp.float32),
            pltpu.VMEM_SHARED((N, Dp), jnp.float32),
            pltpu.SemaphoreType.DMA,
            pltpu.SemaphoreType.DMA,
        ],
    )
    def seg(tab_ref, src_ref, dst_ref, out_ref,
            sbufa, dbufa, rowsa, sbufb, dbufb, rowsb, acc, sem_a, sem_b):
        c = lax.axis_index("c")
        s = lax.axis_index("s")
        wid = c * 16 + s
        nu = NBASE + jnp.where(wid < NEXTRA, 1, 0)

        def zrow(i, carry):
            for kk in range(Dp // 16):
                rowsa[0, i, pl.ds(kk * 16, 16)] = jnp.zeros((16,), jnp.float32)
            return carry
        lax.fori_loop(0, CH, zrow, 0)

        zbase = s * RPT
        for i in range(RPT // CH):
            pltpu.sync_copy(rowsa.at[0], acc.at[pl.ds(zbase + i * CH, CH)])
        rem = RPT - (RPT // CH) * CH
        pltpu.sync_copy(rowsa.at[0, pl.ds(0, rem)],
                        acc.at[pl.ds(zbase + (RPT // CH) * CH, rem)])
        plsc.subcore_barrier()

        def prefetch(t, sbuf, dbuf, rows, sem):
            cb = (wid + t * NW) * UC
            pltpu.sync_copy(src_ref.at[pl.ds(cb, UC)], sbuf)
            for j in range(UC):
                pltpu.sync_copy(dst_ref.at[pl.ds((cb + j) * CH, CH)], dbuf[j])
                pltpu.async_copy(tab_ref.at[sbuf.at[j]], rows.at[j], sem)

        def process(dbuf, rows, sem):
            for j in range(UC):
                pltpu.make_async_copy(tab_ref.at[dbuf[j]],
                                      rows.at[j], sem).wait()
                pltpu.sync_copy(rows.at[j], acc.at[dbuf[j]], add=True)

        prefetch(0, sbufa, dbufa, rowsa, sem_a)
        prefetch(1, sbufb, dbufb, rowsb, sem_b)

        def body(i, carry):
            @pl.when(2 * i < nu)
            def _():
                process(dbufa, rowsa, sem_a)

            @pl.when(2 * i + 2 < nu)
            def _():
                prefetch(2 * i + 2, sbufa, dbufa, rowsa, sem_a)

            @pl.when(2 * i + 1 < nu)
            def _():
                process(dbufb, rowsb, sem_b)

            @pl.when(2 * i + 3 < nu)
            def _():
                prefetch(2 * i + 3, sbufb, dbufb, rowsb, sem_b)
            return carry
        lax.fori_loop(0, (NMAX + 1) // 2, body, 0)

        plsc.subcore_barrier()
        pltpu.sync_copy(acc.at[pl.ds(s * RPT, RPT)],
                        out_ref.at[c, pl.ds(s * RPT, RPT)])

    return seg


_segsum16 = _make_segsum(16)
_segsum144 = _make_segsum(144)


def _dense_body(Din, parts_ref, f_ref, wl_ref, bl_ref, wr_ref, w_ref,
                h_ref, u_ref):
    p = parts_ref[0] + parts_ref[1]
    deg = jnp.maximum(p[:, Din:Din + 1], 1.0)
    mean = p[:, :Din] / deg
    xin = f_ref[:, :Din]
    h = mean @ wl_ref[...] + bl_ref[...] + xin @ wr_ref[...]
    h = jnp.maximum(h, 0.0)
    w = w_ref[...]
    u = (h @ w) * lax.rsqrt(jnp.sum(w * w))
    keep = f_ref[:, Din:Din + 1]
    h_ref[...] = h
    u_ref[...] = jnp.where(keep > 0, u, NEGINF)


def _dense(parts, F, Wl, bl, Wr, w, Din):
    Dp = F.shape[1]
    return pl.pallas_call(
        functools.partial(_dense_body, Din),
        grid=(GRID,),
        in_specs=[
            pl.BlockSpec((2, RB, Dp), lambda i: (0, i, 0)),
            pl.BlockSpec((RB, Dp), lambda i: (i, 0)),
            pl.BlockSpec(Wl.shape, lambda i: (0, 0)),
            pl.BlockSpec(bl.shape, lambda i: (0, 0)),
            pl.BlockSpec(Wr.shape, lambda i: (0, 0)),
            pl.BlockSpec(w.shape, lambda i: (0, 0)),
        ],
        out_specs=[
            pl.BlockSpec((RB, H), lambda i: (i, 0)),
            pl.BlockSpec((RB, 1), lambda i: (i, 0)),
        ],
        out_shape=[
            jax.ShapeDtypeStruct((N, H), jnp.float32),
            jax.ShapeDtypeStruct((N, 1), jnp.float32),
        ],
    )(parts, F, Wl, bl, Wr, w)


def _select_body(k, u_ref, g_ref, keep_ref):
    u = u_ref[...]
    b = lax.bitcast_convert_type(u, jnp.uint32)
    keys = jnp.where(u < 0, ~b, b | jnp.uint32(0x80000000))

    def it(i, prefix):
        shift = (jnp.uint32(31) - i.astype(jnp.uint32))
        cand = prefix | jnp.left_shift(jnp.uint32(1), shift)
        cnt = jnp.sum((keys >= cand).astype(jnp.int32))
        return jnp.where(cnt >= k, cand, prefix)

    thr = lax.fori_loop(0, 32, it, jnp.uint32(0))
    keep = (keys >= thr).astype(jnp.float32)
    g_ref[...] = jnp.tanh(u) * keep
    keep_ref[...] = keep


def _select(upad, k):
    return pl.pallas_call(
        functools.partial(_select_body, k),
        out_shape=[
            jax.ShapeDtypeStruct((NPAD // 128, 128), jnp.float32),
            jax.ShapeDtypeStruct((NPAD // 128, 128), jnp.float32),
        ],
    )(upad)


def _asm_body(h_ref, g_ref, keep_ref, f_ref, ro_ref):
    i = pl.program_id(0)
    g = g_ref[...]
    kp = keep_ref[...]
    hm = h_ref[...] * g
    f_ref[...] = jnp.concatenate(
        [hm, kp, jnp.zeros((RB, 15), jnp.float32)], axis=1)
    mx = jnp.max(jnp.where(kp > 0, hm, NEGINF), axis=0, keepdims=True)
    sm = jnp.sum(hm, axis=0, keepdims=True)

    @pl.when(i == 0)
    def _():
        ro_ref[...] = jnp.concatenate([mx, sm], axis=0)

    @pl.when(i > 0)
    def _():
        ro_ref[0:1, :] = jnp.maximum(ro_ref[0:1, :], mx)
        ro_ref[1:2, :] = ro_ref[1:2, :] + sm


def _assemble(h, g, kp):
    return pl.pallas_call(
        _asm_body,
        grid=(GRID,),
        in_specs=[
            pl.BlockSpec((RB, H), lambda i: (i, 0)),
            pl.BlockSpec((RB, 1), lambda i: (i, 0)),
            pl.BlockSpec((RB, 1), lambda i: (i, 0)),
        ],
        out_specs=[
            pl.BlockSpec((RB, 144), lambda i: (i, 0)),
            pl.BlockSpec((2, H), lambda i: (0, 0)),
        ],
        out_shape=[
            jax.ShapeDtypeStruct((N, 144), jnp.float32),
            jax.ShapeDtypeStruct((2, H), jnp.float32),
        ],
    )(h, g, kp)


def _mlp_body(z_ref, w1, b1, w2, b2, w3, b3, out_ref):
    z = z_ref[...]
    z = jnp.maximum(z @ w1[...] + b1[...], 0.0)
    z = jnp.maximum(z @ w2[...] + b2[...], 0.0)
    z = z @ w3[...] + b3[...]
    out_ref[...] = 1.0 / (1.0 + jnp.exp(-z))


def _mlp(z, W1, b1, W2, b2, W3, b3):
    return pl.pallas_call(
        _mlp_body,
        out_shape=jax.ShapeDtypeStruct((1, 1), jnp.float32),
    )(z, W1, b1, W2, b2, W3, b3)


def kernel(x, edge_index, batch, emb,
           conv1_Wl, conv1_bl, conv1_Wr, pool1_w,
           conv2_Wl, conv2_bl, conv2_Wr, pool2_w,
           conv3_Wl, conv3_bl, conv3_Wr, pool3_w,
           lin1_W, lin1_b, lin2_W, lin2_b, lin3_W, lin3_b):
    V = emb.shape[0]
    xi = x[:, 0]
    src2 = jnp.asarray(edge_index[0]).reshape(E // CH, CH)
    dst = jnp.asarray(edge_index[1])

    emb_pad = jnp.concatenate(
        [emb, jnp.ones((V, 1), jnp.float32), jnp.zeros((V, 6), jnp.float32)],
        axis=1)

    F = _he_gather(emb_pad, xi)
    Din = 9
    seg = _segsum16
    layers = [
        (conv1_Wl, conv1_bl, conv1_Wr, pool1_w, 8000),
        (conv2_Wl, conv2_bl, conv2_Wr, pool2_w, 6400),
        (conv3_Wl, conv3_bl, conv3_Wr, pool3_w, 5120),
    ]
    zparts = []
    for Wl, bl, Wr, pw, k in layers:
        parts = seg(F, src2, dst)
        h, u = _dense(parts, F, Wl, bl[None, :], Wr, pw[:, None], Din)
        upad = jnp.concatenate(
            [u[:, 0], jnp.full((NPAD - N,), NEGINF, jnp.float32)]
        ).reshape(NPAD // 128, 128)
        g80, k80 = _select(upad, k)
        g = g80.reshape(NPAD)[:N][:, None]
        kp = k80.reshape(NPAD)[:N][:, None]
        F, ro = _assemble(h, g, kp)
        zparts.append(jnp.concatenate([ro[0], ro[1] / k]))
        Din = 128
        seg = _segsum144

    z = (zparts[0] + zparts[1] + zparts[2])[None, :]
    out = _mlp(z, lin1_W, lin1_b[None, :], lin2_W, lin2_b[None, :],
               lin3_W, lin3_b[None, :])
    return out[:, 0]


# speedup vs baseline: 22.9625x; 1.1430x over previous
import functools

import jax
import jax.numpy as jnp
from jax import lax
from jax.experimental import pallas as pl
from jax.experimental.pallas import tpu as pltpu
from jax.experimental.pallas import tpu_sc as plsc

N = 10000
E = 320000
H = 128
NW = 32
EPW = E // NW
CH = 128
NFULL = EPW // CH
TAIL = EPW - NFULL * CH
RPT = N // 16
RB = 1000
GRID = N // RB
NPAD = 10240
NEGINF = float("-inf")


def _sc_mesh():
    return plsc.VectorSubcoreMesh(core_axis_name="c", subcore_axis_name="s")


def _he_gather(emb_pad, xi):
    RW = 312
    GC = 104

    @functools.partial(
        pl.kernel,
        out_type=jax.ShapeDtypeStruct((N, 16), jnp.float32),
        mesh=_sc_mesh(),
        compiler_params=pltpu.CompilerParams(use_tc_tiling_on_sc=False),
        scratch_types=[
            pltpu.VMEM((GC,), jnp.int32),
            pltpu.VMEM((GC, 16), jnp.float32),
            pltpu.VMEM((16,), jnp.int32),
            pltpu.VMEM((16, 16), jnp.float32),
            pltpu.SemaphoreType.DMA,
        ],
    )
    def hek(emb_ref, xi_ref, out_ref, idx, rows, idxt, rowst, sem):
        c = lax.axis_index("c")
        s = lax.axis_index("s")
        wid = c * 16 + s
        base = wid * RW
        for i in range(RW // GC):
            off = base + i * GC
            pltpu.sync_copy(xi_ref.at[pl.ds(off, GC)], idx)
            pltpu.async_copy(emb_ref.at[idx], rows, sem).wait()
            pltpu.sync_copy(rows, out_ref.at[pl.ds(off, GC)])

        @pl.when(wid == 0)
        def _():
            pltpu.sync_copy(xi_ref.at[pl.ds(NW * RW, 16)], idxt)
            pltpu.async_copy(emb_ref.at[idxt], rowst, sem).wait()
            pltpu.sync_copy(rowst, out_ref.at[pl.ds(NW * RW, 16)])

    return hek(emb_pad, xi)


def _make_segsum(Dp):
    UC = 4 if Dp == 16 else 1
    NUNITS = E // (UC * CH)
    NBASE = NUNITS // NW
    NEXTRA = NUNITS - NBASE * NW
    NMAX = NBASE + (1 if NEXTRA else 0)

    @functools.partial(
        pl.kernel,
        out_type=jax.ShapeDtypeStruct((2, N, Dp), jnp.float32),
        mesh=_sc_mesh(),
        compiler_params=pltpu.CompilerParams(use_tc_tiling_on_sc=False),
        scratch_types=[
            pltpu.VMEM((UC, CH), jnp.int32),
            [pltpu.VMEM((CH,), jnp.int32) for _ in range(UC)],
            pltpu.VMEM((UC, CH, Dp), jnp.float32),
            pltpu.VMEM((UC, CH), jnp.int32),
            [pltpu.VMEM((CH,), jnp.int32) for _ in range(UC)],
            pltpu.VMEM((UC, CH, Dp), jnp.float32),
            pltpu.VMEM_SHARED((N, Dp), jnp.float32),
            pltpu.SemaphoreType.DMA,
            pltpu.SemaphoreType.DMA,
        ],
    )
    def seg(tab_ref, src_ref, dst_ref, out_ref,
            sbufa, dbufa, rowsa, sbufb, dbufb, rowsb, acc, sem_a, sem_b):
        c = lax.axis_index("c")
        s = lax.axis_index("s")
        wid = c * 16 + s
        nu = NBASE + jnp.where(wid < NEXTRA, 1, 0)

        def zrow(i, carry):
            for kk in range(Dp // 16):
                rowsa[0, i, pl.ds(kk * 16, 16)] = jnp.zeros((16,), jnp.float32)
            return carry
        lax.fori_loop(0, CH, zrow, 0)

        zbase = s * RPT
        for i in range(RPT // CH):
            pltpu.sync_copy(rowsa.at[0], acc.at[pl.ds(zbase + i * CH, CH)])
        rem = RPT - (RPT // CH) * CH
        pltpu.sync_copy(rowsa.at[0, pl.ds(0, rem)],
                        acc.at[pl.ds(zbase + (RPT // CH) * CH, rem)])
        plsc.subcore_barrier()

        def prefetch(t, sbuf, dbuf, rows, sem):
            cb = (wid + t * NW) * UC
            pltpu.sync_copy(src_ref.at[pl.ds(cb, UC)], sbuf)
            for j in range(UC):
                pltpu.sync_copy(dst_ref.at[pl.ds((cb + j) * CH, CH)], dbuf[j])
                pltpu.async_copy(tab_ref.at[sbuf.at[j]], rows.at[j], sem)

        def process(dbuf, rows, sem):
            for j in range(UC):
                pltpu.make_async_copy(tab_ref.at[dbuf[j]],
                                      rows.at[j], sem).wait()
                pltpu.sync_copy(rows.at[j], acc.at[dbuf[j]], add=True)

        prefetch(0, sbufa, dbufa, rowsa, sem_a)
        prefetch(1, sbufb, dbufb, rowsb, sem_b)

        def body(i, carry):
            @pl.when(2 * i < nu)
            def _():
                process(dbufa, rowsa, sem_a)

            @pl.when(2 * i + 2 < nu)
            def _():
                prefetch(2 * i + 2, sbufa, dbufa, rowsa, sem_a)

            @pl.when(2 * i + 1 < nu)
            def _():
                process(dbufb, rowsb, sem_b)

            @pl.when(2 * i + 3 < nu)
            def _():
                prefetch(2 * i + 3, sbufb, dbufb, rowsb, sem_b)
            return carry
        lax.fori_loop(0, (NMAX + 1) // 2, body, 0)

        plsc.subcore_barrier()
        pltpu.sync_copy(acc.at[pl.ds(s * RPT, RPT)],
                        out_ref.at[c, pl.ds(s * RPT, RPT)])

    return seg


_segsum16 = _make_segsum(16)
_segsum144 = _make_segsum(144)


def _layer_common(Din, k, parts_ref, f_ref, wl_ref, bl_ref, wr_ref, w_ref):
    p = parts_ref[0] + parts_ref[1]
    deg = jnp.maximum(p[:, Din:Din + 1], 1.0)
    mean = p[:, :Din] / deg
    xin = f_ref[:, :Din]
    h = mean @ wl_ref[...] + bl_ref[...] + xin @ wr_ref[...]
    h = jnp.maximum(h, 0.0)
    w = w_ref[...]
    wt = jnp.transpose(w)
    ut = lax.dot_general(wt, h, (((1,), (1,)), ((), ())),
                         preferred_element_type=jnp.float32)
    ut = ut * lax.rsqrt(jnp.sum(w * w))
    keeprow = jnp.transpose(f_ref[:, Din:Din + 1])
    ueff = jnp.where(keeprow > 0, ut, NEGINF)
    b = lax.bitcast_convert_type(ueff, jnp.uint32)
    keys = jnp.where(ueff < 0, ~b, b | jnp.uint32(0x80000000))

    def it(i, prefix):
        shift = jnp.uint32(31) - i.astype(jnp.uint32)
        cand = prefix | jnp.left_shift(jnp.uint32(1), shift)
        cnt = jnp.sum((keys >= cand).astype(jnp.int32))
        return jnp.where(cnt >= k, cand, prefix)

    thr = lax.fori_loop(0, 32, it, jnp.uint32(0))
    keepn = (keys >= thr).astype(jnp.float32)
    grow = jnp.tanh(ut) * keepn
    hm = h * jnp.transpose(grow)
    kp = jnp.transpose(keepn)
    mx = jnp.max(jnp.where(kp > 0, hm, NEGINF), axis=0, keepdims=True)
    sm = jnp.sum(hm, axis=0, keepdims=True) * (1.0 / k)
    z = jnp.concatenate([mx, sm], axis=1)
    return hm, kp, z


def _layer_body(Din, k, parts_ref, f_ref, wl_ref, bl_ref, wr_ref, w_ref,
                fout_ref, z_ref):
    hm, kp, z = _layer_common(Din, k, parts_ref, f_ref, wl_ref, bl_ref,
                              wr_ref, w_ref)
    fout_ref[...] = jnp.concatenate(
        [hm, kp, jnp.zeros((N, 15), jnp.float32)], axis=1)
    z_ref[...] = z


def _layer(parts, F, Wl, bl, Wr, w, Din, k):
    return pl.pallas_call(
        functools.partial(_layer_body, Din, k),
        out_shape=[
            jax.ShapeDtypeStruct((N, 144), jnp.float32),
            jax.ShapeDtypeStruct((1, 2 * H), jnp.float32),
        ],
    )(parts, F, Wl, bl, Wr, w)


def _layer3_body(Din, k, parts_ref, f_ref, wl_ref, bl_ref, wr_ref, w_ref,
                 z1_ref, z2_ref, w1_ref, b1_ref, w2_ref, b2_ref,
                 w3_ref, b3_ref, out_ref):
    _, _, z3 = _layer_common(Din, k, parts_ref, f_ref, wl_ref, bl_ref,
                             wr_ref, w_ref)
    z = z1_ref[...] + z2_ref[...] + z3
    z = jnp.maximum(z @ w1_ref[...] + b1_ref[...], 0.0)
    z = jnp.maximum(z @ w2_ref[...] + b2_ref[...], 0.0)
    z = z @ w3_ref[...] + b3_ref[...]
    out_ref[...] = 1.0 / (1.0 + jnp.exp(-z))


def _layer3(parts, F, Wl, bl, Wr, w, z1, z2, mlp, Din, k):
    W1, b1, W2, b2, W3, b3 = mlp
    return pl.pallas_call(
        functools.partial(_layer3_body, Din, k),
        out_shape=jax.ShapeDtypeStruct((1, 1), jnp.float32),
    )(parts, F, Wl, bl, Wr, w, z1, z2, W1, b1, W2, b2, W3, b3)


def kernel(x, edge_index, batch, emb,
           conv1_Wl, conv1_bl, conv1_Wr, pool1_w,
           conv2_Wl, conv2_bl, conv2_Wr, pool2_w,
           conv3_Wl, conv3_bl, conv3_Wr, pool3_w,
           lin1_W, lin1_b, lin2_W, lin2_b, lin3_W, lin3_b):
    V = emb.shape[0]
    xi = x[:, 0]
    src2 = jnp.asarray(edge_index[0]).reshape(E // CH, CH)
    dst = jnp.asarray(edge_index[1])

    emb_pad = jnp.concatenate(
        [emb, jnp.ones((V, 1), jnp.float32), jnp.zeros((V, 6), jnp.float32)],
        axis=1)

    F = _he_gather(emb_pad, xi)
    parts = _segsum16(F, src2, dst)
    F, z1 = _layer(parts, F, conv1_Wl, conv1_bl[None, :], conv1_Wr,
                   pool1_w[:, None], 9, 8000)
    parts = _segsum144(F, src2, dst)
    F, z2 = _layer(parts, F, conv2_Wl, conv2_bl[None, :], conv2_Wr,
                   pool2_w[:, None], 128, 6400)
    parts = _segsum144(F, src2, dst)
    out = _layer3(parts, F, conv3_Wl, conv3_bl[None, :], conv3_Wr,
                  pool3_w[:, None], z1, z2,
                  (lin1_W, lin1_b[None, :], lin2_W, lin2_b[None, :],
                   lin3_W, lin3_b[None, :]), 128, 5120)
    return out[:, 0]
